# baseline (speedup 1.0000x reference)
import jax
import jax.numpy as jnp
from jax import lax
from jax.experimental import pallas as pl
from jax.experimental.pallas import tpu as pltpu
from jax.experimental.pallas import tpu_sc as plsc

SEQ = 200
BATCH = 4096
EMBED = 64
NC = 2
NS = 16
NW = NC * NS
BPW = BATCH // NW
G = 4
NITER = SEQ // (2 * G)
LANES = 16
CH = EMBED // LANES


def _body(idx_hbm, table_hbm, fcw_hbm, fcb_hbm, out_hbm,
          idx_v, acc_v,
          a0, a1, a2, a3, b0, b1, b2, b3,
          out_v, fcw_v, fcb_v, sem_a, sem_b):
    wid = lax.axis_index("s") * NC + lax.axis_index("c")
    bufs_a = (a0, a1, a2, a3)
    bufs_b = (b0, b1, b2, b3)

    pltpu.sync_copy(fcw_hbm, fcw_v)
    pltpu.sync_copy(fcb_hbm, fcb_v)
    pltpu.sync_copy(idx_hbm.at[wid], idx_v)

    def fire(bufs, sem, base):
        for k in range(G):
            pltpu.async_copy(table_hbm.at[idx_v.at[base + k]], bufs[k], sem)

    def drain(bufs, sem, base):
        for k in range(G):
            pltpu.make_async_copy(
                table_hbm.at[idx_v.at[base + k]], bufs[k], sem).wait()

    zeros = jnp.zeros((LANES,), jnp.float32)

    @pl.loop(0, BPW)
    def _(i):
        for c in range(CH):
            acc_v[pl.ds(i * EMBED + c * LANES, LANES)] = zeros

    def accumulate(bufs):
        @pl.loop(0, BPW)
        def _(i):
            for c in range(CH):
                sl = pl.ds(c * LANES, LANES)
                asl = pl.ds(i * EMBED + c * LANES, LANES)
                s = ((bufs[0][i, sl] + bufs[1][i, sl])
                     + (bufs[2][i, sl] + bufs[3][i, sl]))
                acc_v[asl] = acc_v[asl] + s

    fire(bufs_a, sem_a, 0)
    fire(bufs_b, sem_b, G)

    @pl.loop(0, NITER - 1)
    def _(it):
        base = it * (2 * G)
        drain(bufs_a, sem_a, base)
        accumulate(bufs_a)
        fire(bufs_a, sem_a, base + 2 * G)
        drain(bufs_b, sem_b, base + G)
        accumulate(bufs_b)
        fire(bufs_b, sem_b, base + 3 * G)

    last = (NITER - 1) * (2 * G)
    drain(bufs_a, sem_a, last)
    accumulate(bufs_a)
    drain(bufs_b, sem_b, last + G)
    accumulate(bufs_b)

    bias0 = fcb_v[0, :]
    bias1 = fcb_v[1, :]

    @pl.loop(0, BPW // LANES)
    def _(g):
        rows = g * LANES + lax.iota(jnp.int32, LANES)

        def dot_step(d, carry):
            o0, o1 = carry
            col = plsc.load_gather(acc_v, [rows * EMBED + d])
            return (o0 + col * fcw_v[0, d, :], o1 + col * fcw_v[1, d, :])

        o0, o1 = lax.fori_loop(0, EMBED, dot_step, (zeros, zeros))
        o0 = o0 + bias0
        o1 = o1 + bias1
        m = jnp.maximum(o0, o1)
        e0 = jnp.exp(o0 - m)
        e1 = jnp.exp(o1 - m)
        tot = e0 + e1
        sl = pl.ds(g * LANES, LANES)
        out_v[0, sl] = e0 / tot
        out_v[1, sl] = e1 / tot

    pltpu.sync_copy(out_v, out_hbm.at[:, pl.ds(wid * BPW, BPW)])


_sc_call = pl.kernel(
    _body,
    out_type=jax.ShapeDtypeStruct((2, BATCH), jnp.float32),
    mesh=plsc.VectorSubcoreMesh(core_axis_name="c", subcore_axis_name="s"),
    scratch_types=[
        pltpu.VMEM((SEQ, BPW), jnp.int32),
        pltpu.VMEM((BPW * EMBED,), jnp.float32),
    ] + [pltpu.VMEM((BPW, EMBED), jnp.float32) for _ in range(2 * G)]
    + [
        pltpu.VMEM((2, BPW), jnp.float32),
        pltpu.VMEM((2, EMBED, LANES), jnp.float32),
        pltpu.VMEM((2, LANES), jnp.float32),
        pltpu.SemaphoreType.DMA,
        pltpu.SemaphoreType.DMA,
    ],
    compiler_params=pltpu.CompilerParams(
        needs_layout_passes=False, use_tc_tiling_on_sc=False),
)


@jax.jit
def kernel(input, embeddings, fc_w, fc_b):
    idx_r = input.reshape(SEQ, NW, BPW).transpose(1, 0, 2)
    fcw_t = jnp.broadcast_to(
        (fc_w.T * (1.0 / SEQ))[:, :, None], (2, EMBED, LANES))
    fcb_p = jnp.broadcast_to(fc_b[:, None], (2, LANES))
    out2 = _sc_call(idx_r, embeddings, fcw_t, fcb_p)
    return out2.T
